# baseline (speedup 1.0000x reference)
import functools

import jax
import jax.numpy as jnp
from jax import lax
from jax.experimental import pallas as pl
from jax.experimental.pallas import tpu as pltpu
from jax.experimental.pallas import tpu_sc as plsc

BATCH = 1024
HIST = 200
BUCKETS = 1000000
D = 48
DIM = 512
TOK = BATCH * HIST

NC = 2
NS = 16
NW = NC * NS
TPW = TOK // NW
GCHUNK = 128
OCHUNK = 256
NGATHER = OCHUNK // GCHUNK
NOUT = TPW // OCHUNK

BT = 2048
G = TOK // BT


def _sc_hash_gather(idx_flat, idx_prev, tblp):
    mesh = plsc.VectorSubcoreMesh(core_axis_name="c", subcore_axis_name="s")

    @functools.partial(
        pl.kernel,
        mesh=mesh,
        out_type=(
            jax.ShapeDtypeStruct((TOK, 128), jnp.int32),
            jax.ShapeDtypeStruct((TOK,), jnp.int32),
        ),
        scratch_types=[
            pltpu.VMEM((TPW,), jnp.int32),
            pltpu.VMEM((TPW,), jnp.int32),
            pltpu.VMEM((TPW,), jnp.int32),
            pltpu.VMEM((TPW,), jnp.int32),
            pltpu.VMEM((OCHUNK, 128), jnp.int32),
            pltpu.VMEM((OCHUNK, 128), jnp.int32),
            pltpu.SemaphoreType.DMA,
            pltpu.SemaphoreType.DMA,
        ],
        compiler_params=pltpu.CompilerParams(use_tc_tiling_on_sc=False),
    )
    def k(idx_hbm, prev_hbm, tbl_hbm, rows_hbm, big_hbm,
          idx_v, prev_v, big_v, qid_v, buf_a, buf_b, sem_a, sem_b):
        wid = lax.axis_index("s") * NC + lax.axis_index("c")
        base = wid * TPW
        pltpu.sync_copy(idx_hbm.at[pl.ds(base, TPW)], idx_v)
        pltpu.sync_copy(prev_hbm.at[pl.ds(base, TPW)], prev_v)

        lanes = lax.iota(jnp.int32, 16)

        def hash_body(i, _):
            off = i * 16
            pos = off + lanes
            cur = idx_v[pl.ds(off, 16)]
            prev = prev_v[pl.ds(off, 16)]
            b = (prev * 10007 + cur) % BUCKETS
            b = jnp.where(pos % HIST == 0, 0, b)
            big_v[pl.ds(off, 16)] = b
            qid_v[pl.ds(off, 16)] = b >> 1
            return 0

        lax.fori_loop(0, TPW // 16, hash_body, 0)

        pltpu.sync_copy(big_v, big_hbm.at[pl.ds(base, TPW)])

        bufs = (buf_a, buf_b)
        sems = (sem_a, sem_b)

        def fire(c, buf, sem):
            cps = []
            for j in range(NGATHER):
                srow = c * OCHUNK + j * GCHUNK
                cps.append(pltpu.async_copy(
                    tbl_hbm.at[qid_v.at[pl.ds(srow, GCHUNK)]],
                    buf.at[pl.ds(j * GCHUNK, GCHUNK)],
                    sem))
            return cps

        inflight = {0: fire(0, bufs[0], sems[0])}
        for c in range(NOUT):
            if c + 1 < NOUT:
                inflight[c + 1] = fire(c + 1, bufs[(c + 1) % 2], sems[(c + 1) % 2])
            for cp in inflight.pop(c):
                cp.wait()
            pltpu.sync_copy(bufs[c % 2],
                            rows_hbm.at[pl.ds(base + c * OCHUNK, OCHUNK)])

    return k(idx_flat, idx_prev, tblp)


def _f16_bits_to_f32(v):
    sign = (v & 0x8000) << 16
    mag = v & 0x7FFF
    bits = sign | ((mag << 13) + 0x38000000)
    bits = jnp.where((v & 0x7C00) == 0, sign, bits)
    return lax.bitcast_convert_type(bits, jnp.float32)


def _tc_project(rows32, bigs3, wt):
    def body(r_ref, b_ref, w_ref, o_ref):
        pcol = (b_ref[...] & 1).reshape(BT, 1)
        w = r_ref[...][:, :D]
        sel = jnp.where(pcol == 0, w & 0xFFFF, (w >> 16) & 0xFFFF)
        h = _f16_bits_to_f32(sel)
        o_ref[...] = jnp.dot(h, w_ref[...], preferred_element_type=jnp.float32)

    return pl.pallas_call(
        body,
        grid=(G,),
        in_specs=[
            pl.BlockSpec((BT, 128), lambda i: (i, 0)),
            pl.BlockSpec((1, 1, BT), lambda i: (i, 0, 0)),
            pl.BlockSpec((D, DIM), lambda i: (0, 0)),
        ],
        out_specs=pl.BlockSpec((BT, DIM), lambda i: (i, 0)),
        out_shape=jax.ShapeDtypeStruct((TOK, DIM), jnp.float32),
    )(rows32, bigs3, wt)


def kernel(idx, table, W):
    idx_flat = idx.reshape(-1)
    idx_prev = jnp.concatenate([idx_flat[:1], idx_flat[:-1]])
    tpair = lax.bitcast_convert_type(
        table.reshape(BUCKETS // 2, 2, D).transpose(0, 2, 1), jnp.int32)
    tblp = jnp.pad(tpair, ((0, 0), (0, 128 - D)))
    rows32, bigs = _sc_hash_gather(idx_flat, idx_prev, tblp)
    out = _tc_project(rows32, bigs.reshape(G, 1, BT), W.T)
    return out.reshape(BATCH, HIST, DIM)


# speedup vs baseline: 3.9234x; 3.9234x over previous
import functools

import jax
import jax.numpy as jnp
from jax import lax
from jax.experimental import pallas as pl
from jax.experimental.pallas import tpu as pltpu
from jax.experimental.pallas import tpu_sc as plsc

BATCH = 1024
HIST = 200
BUCKETS = 1000000
D = 48
DIM = 512
TOK = BATCH * HIST

NC = 2
NS = 16
NW = NC * NS
TPW = TOK // NW
GCHUNK = 128
OCHUNK = 256
NGATHER = OCHUNK // GCHUNK
NOUT = TPW // OCHUNK

BR = 1600
GP = BUCKETS // BR

BT = 2048
G = TOK // BT


def _tc_prep(tbl_bf):
    def body(t_ref, o_ref):
        w = t_ref.bitcast(jnp.int32)[...]
        o_ref[:, :D] = w

    return pl.pallas_call(
        body,
        grid=(GP,),
        in_specs=[pl.BlockSpec((BR, D), lambda i: (i, 0))],
        out_specs=pl.BlockSpec((BR // 2, 128), lambda i: (i, 0)),
        out_shape=jax.ShapeDtypeStruct((BUCKETS // 2, 128), jnp.int32),
    )(tbl_bf)


def _sc_hash_gather(idx_flat, idx_prev, tblp):
    mesh = plsc.VectorSubcoreMesh(core_axis_name="c", subcore_axis_name="s")

    @functools.partial(
        pl.kernel,
        mesh=mesh,
        out_type=(
            jax.ShapeDtypeStruct((TOK, 128), jnp.int32),
            jax.ShapeDtypeStruct((TOK,), jnp.int32),
        ),
        scratch_types=[
            pltpu.VMEM((TPW,), jnp.int32),
            pltpu.VMEM((TPW,), jnp.int32),
            pltpu.VMEM((TPW,), jnp.int32),
            pltpu.VMEM((TPW,), jnp.int32),
            pltpu.VMEM((OCHUNK, 128), jnp.int32),
            pltpu.VMEM((OCHUNK, 128), jnp.int32),
            pltpu.SemaphoreType.DMA,
            pltpu.SemaphoreType.DMA,
        ],
        compiler_params=pltpu.CompilerParams(use_tc_tiling_on_sc=False),
    )
    def k(idx_hbm, prev_hbm, tbl_hbm, rows_hbm, big_hbm,
          idx_v, prev_v, big_v, qid_v, buf_a, buf_b, sem_a, sem_b):
        wid = lax.axis_index("s") * NC + lax.axis_index("c")
        base = wid * TPW
        pltpu.sync_copy(idx_hbm.at[pl.ds(base, TPW)], idx_v)
        pltpu.sync_copy(prev_hbm.at[pl.ds(base, TPW)], prev_v)

        lanes = lax.iota(jnp.int32, 16)

        def hash_body(i, _):
            off = i * 16
            pos = off + lanes
            cur = idx_v[pl.ds(off, 16)]
            prev = prev_v[pl.ds(off, 16)]
            b = (prev * 10007 + cur) % BUCKETS
            b = jnp.where(pos % HIST == 0, 0, b)
            big_v[pl.ds(off, 16)] = b
            qid_v[pl.ds(off, 16)] = b >> 1
            return 0

        lax.fori_loop(0, TPW // 16, hash_body, 0)

        pltpu.sync_copy(big_v, big_hbm.at[pl.ds(base, TPW)])

        bufs = (buf_a, buf_b)
        sems = (sem_a, sem_b)

        def fire(c, buf, sem):
            cps = []
            for j in range(NGATHER):
                srow = c * OCHUNK + j * GCHUNK
                cps.append(pltpu.async_copy(
                    tbl_hbm.at[qid_v.at[pl.ds(srow, GCHUNK)]],
                    buf.at[pl.ds(j * GCHUNK, GCHUNK)],
                    sem))
            return cps

        inflight = {0: fire(0, bufs[0], sems[0])}
        for c in range(NOUT):
            if c + 1 < NOUT:
                inflight[c + 1] = fire(c + 1, bufs[(c + 1) % 2], sems[(c + 1) % 2])
            for cp in inflight.pop(c):
                cp.wait()
            pltpu.sync_copy(bufs[c % 2],
                            rows_hbm.at[pl.ds(base + c * OCHUNK, OCHUNK)])

    return k(idx_flat, idx_prev, tblp)


def _f16_bits_to_f32(v):
    sign = (v & 0x8000) << 16
    mag = v & 0x7FFF
    bits = sign | ((mag << 13) + 0x38000000)
    bits = jnp.where((v & 0x7C00) == 0, sign, bits)
    return lax.bitcast_convert_type(bits, jnp.float32)


def _tc_project(rows32, bigs3, wt):
    def body(r_ref, b_ref, w_ref, o_ref):
        pcol = (b_ref[...] & 1).reshape(BT, 1)
        w = r_ref[...][:, :D]
        sel = jnp.where(pcol == 0, w & 0xFFFF, (w >> 16) & 0xFFFF)
        h = _f16_bits_to_f32(sel)
        o_ref[...] = jnp.dot(h, w_ref[...], preferred_element_type=jnp.float32)

    return pl.pallas_call(
        body,
        grid=(G,),
        in_specs=[
            pl.BlockSpec((BT, 128), lambda i: (i, 0)),
            pl.BlockSpec((1, 1, BT), lambda i: (i, 0, 0)),
            pl.BlockSpec((D, DIM), lambda i: (0, 0)),
        ],
        out_specs=pl.BlockSpec((BT, DIM), lambda i: (i, 0)),
        out_shape=jax.ShapeDtypeStruct((TOK, DIM), jnp.float32),
    )(rows32, bigs3, wt)


def kernel(idx, table, W):
    idx_flat = idx.reshape(-1)
    idx_prev = jnp.concatenate([idx_flat[:1], idx_flat[:-1]])
    tblp = _tc_prep(lax.bitcast_convert_type(table, jnp.bfloat16))
    rows32, bigs = _sc_hash_gather(idx_flat, idx_prev, tblp)
    out = _tc_project(rows32, bigs.reshape(G, 1, BT), W.T)
    return out.reshape(BATCH, HIST, DIM)


# speedup vs baseline: 5.3766x; 1.3704x over previous
import functools

import jax
import jax.numpy as jnp
from jax import lax
from jax.experimental import pallas as pl
from jax.experimental.pallas import tpu as pltpu
from jax.experimental.pallas import tpu_sc as plsc

BATCH = 1024
HIST = 200
BUCKETS = 1000000
D = 48
DIM = 512
TOK = BATCH * HIST

NC = 2
NS = 16
NW = NC * NS
TPW = TOK // NW
GCHUNK = 128
OCHUNK = 256
NGATHER = OCHUNK // GCHUNK
NOUT = TPW // OCHUNK

BC = 2048
GP = -(-BUCKETS // BC)

BT = 2048
G = TOK // BT


def _tc_prep(tbl_bft):
    def body(t_ref, o_ref):
        x = t_ref[...]
        xt = x.T
        w = pltpu.bitcast(xt, jnp.int32)
        o_ref[:, :D] = w

    return pl.pallas_call(
        body,
        grid=(GP,),
        in_specs=[pl.BlockSpec((D, BC), lambda i: (0, i))],
        out_specs=pl.BlockSpec((BC // 2, 128), lambda i: (i, 0)),
        out_shape=jax.ShapeDtypeStruct((BUCKETS // 2, 128), jnp.int32),
    )(tbl_bft)


def _sc_hash_gather(idx_flat, idx_prev, tblp):
    mesh = plsc.VectorSubcoreMesh(core_axis_name="c", subcore_axis_name="s")

    @functools.partial(
        pl.kernel,
        mesh=mesh,
        out_type=(
            jax.ShapeDtypeStruct((TOK, 128), jnp.int32),
            jax.ShapeDtypeStruct((TOK,), jnp.int32),
        ),
        scratch_types=[
            pltpu.VMEM((TPW,), jnp.int32),
            pltpu.VMEM((TPW,), jnp.int32),
            pltpu.VMEM((TPW,), jnp.int32),
            pltpu.VMEM((TPW,), jnp.int32),
            pltpu.VMEM((OCHUNK, 128), jnp.int32),
            pltpu.VMEM((OCHUNK, 128), jnp.int32),
            pltpu.SemaphoreType.DMA,
            pltpu.SemaphoreType.DMA,
        ],
        compiler_params=pltpu.CompilerParams(use_tc_tiling_on_sc=False),
    )
    def k(idx_hbm, prev_hbm, tbl_hbm, rows_hbm, big_hbm,
          idx_v, prev_v, big_v, qid_v, buf_a, buf_b, sem_a, sem_b):
        wid = lax.axis_index("s") * NC + lax.axis_index("c")
        base = wid * TPW
        pltpu.sync_copy(idx_hbm.at[pl.ds(base, TPW)], idx_v)
        pltpu.sync_copy(prev_hbm.at[pl.ds(base, TPW)], prev_v)

        lanes = lax.iota(jnp.int32, 16)

        def hash_body(i, _):
            off = i * 16
            pos = off + lanes
            cur = idx_v[pl.ds(off, 16)]
            prev = prev_v[pl.ds(off, 16)]
            b = (prev * 10007 + cur) % BUCKETS
            b = jnp.where(pos % HIST == 0, 0, b)
            big_v[pl.ds(off, 16)] = b
            qid_v[pl.ds(off, 16)] = b >> 1
            return 0

        lax.fori_loop(0, TPW // 16, hash_body, 0)

        pltpu.sync_copy(big_v, big_hbm.at[pl.ds(base, TPW)])

        bufs = (buf_a, buf_b)
        sems = (sem_a, sem_b)

        def fire(c, buf, sem):
            cps = []
            for j in range(NGATHER):
                srow = c * OCHUNK + j * GCHUNK
                cps.append(pltpu.async_copy(
                    tbl_hbm.at[qid_v.at[pl.ds(srow, GCHUNK)]],
                    buf.at[pl.ds(j * GCHUNK, GCHUNK)],
                    sem))
            return cps

        inflight = {0: fire(0, bufs[0], sems[0])}
        for c in range(NOUT):
            if c + 1 < NOUT:
                inflight[c + 1] = fire(c + 1, bufs[(c + 1) % 2], sems[(c + 1) % 2])
            for cp in inflight.pop(c):
                cp.wait()
            pltpu.sync_copy(bufs[c % 2],
                            rows_hbm.at[pl.ds(base + c * OCHUNK, OCHUNK)])

    return k(idx_flat, idx_prev, tblp)


def _f16_bits_to_f32(v):
    sign = (v & 0x8000) << 16
    mag = v & 0x7FFF
    bits = sign | ((mag << 13) + 0x38000000)
    bits = jnp.where((v & 0x7C00) == 0, sign, bits)
    return lax.bitcast_convert_type(bits, jnp.float32)


def _tc_project(rows32, bigs3, wt):
    def body(r_ref, b_ref, w_ref, o_ref):
        pcol = (b_ref[...] & 1).reshape(BT, 1)
        w = r_ref[...][:, :D]
        sel = jnp.where(pcol == 0, w & 0xFFFF, (w >> 16) & 0xFFFF)
        h = _f16_bits_to_f32(sel)
        o_ref[...] = jnp.dot(h, w_ref[...], preferred_element_type=jnp.float32)

    return pl.pallas_call(
        body,
        grid=(G,),
        in_specs=[
            pl.BlockSpec((BT, 128), lambda i: (i, 0)),
            pl.BlockSpec((1, 1, BT), lambda i: (i, 0, 0)),
            pl.BlockSpec((D, DIM), lambda i: (0, 0)),
        ],
        out_specs=pl.BlockSpec((BT, DIM), lambda i: (i, 0)),
        out_shape=jax.ShapeDtypeStruct((TOK, DIM), jnp.float32),
    )(rows32, bigs3, wt)


def kernel(idx, table, W):
    idx_flat = idx.reshape(-1)
    idx_prev = jnp.concatenate([idx_flat[:1], idx_flat[:-1]])
    tblp = _tc_prep(lax.bitcast_convert_type(table, jnp.bfloat16).T)
    rows32, bigs = _sc_hash_gather(idx_flat, idx_prev, tblp)
    out = _tc_project(rows32, bigs.reshape(G, 1, BT), W.T)
    return out.reshape(BATCH, HIST, DIM)


# speedup vs baseline: 5.5224x; 1.0271x over previous
import functools

import jax
import jax.numpy as jnp
from jax import lax
from jax.experimental import pallas as pl
from jax.experimental.pallas import tpu as pltpu
from jax.experimental.pallas import tpu_sc as plsc

BATCH = 1024
HIST = 200
BUCKETS = 1000000
D = 48
DIM = 512
TOK = BATCH * HIST

NC = 2
NS = 16
NW = NC * NS
TPW = TOK // NW
GCHUNK = 128
OCHUNK = 256
NGATHER = OCHUNK // GCHUNK
NOUT = TPW // OCHUNK

BC = 2048
GP = -(-BUCKETS // BC)

BT = 2048
G = TOK // BT


def _tc_prep(tbl_bft):
    def body(t_ref, o_ref):
        x = t_ref[...]
        xt = x.T
        w = pltpu.bitcast(xt, jnp.int32)
        o_ref[:, :D] = w

    return pl.pallas_call(
        body,
        grid=(GP,),
        in_specs=[pl.BlockSpec((D, BC), lambda i: (0, i))],
        out_specs=pl.BlockSpec((BC // 2, 128), lambda i: (i, 0)),
        out_shape=jax.ShapeDtypeStruct((BUCKETS // 2, 128), jnp.int32),
    )(tbl_bft)


def _sc_hash_gather(idx_flat, idx_prev, tblp):
    mesh = plsc.VectorSubcoreMesh(core_axis_name="c", subcore_axis_name="s")

    @functools.partial(
        pl.kernel,
        mesh=mesh,
        out_type=(
            jax.ShapeDtypeStruct((TOK, 128), jnp.int32),
            jax.ShapeDtypeStruct((TOK,), jnp.int32),
        ),
        scratch_types=[
            pltpu.VMEM((TPW,), jnp.int32),
            pltpu.VMEM((TPW,), jnp.int32),
            pltpu.VMEM((TPW,), jnp.int32),
            pltpu.VMEM((TPW,), jnp.int32),
            pltpu.VMEM((OCHUNK, 128), jnp.int32),
            pltpu.VMEM((OCHUNK, 128), jnp.int32),
            pltpu.SemaphoreType.DMA,
            pltpu.SemaphoreType.DMA,
        ],
        compiler_params=pltpu.CompilerParams(use_tc_tiling_on_sc=False),
    )
    def k(idx_hbm, prev_hbm, tbl_hbm, rows_hbm, big_hbm,
          idx_v, prev_v, big_v, qid_v, buf_a, buf_b, sem_a, sem_b):
        wid = lax.axis_index("s") * NC + lax.axis_index("c")
        base = wid * TPW
        pltpu.sync_copy(idx_hbm.at[pl.ds(base, TPW)], idx_v)
        pltpu.sync_copy(prev_hbm.at[pl.ds(base, TPW)], prev_v)

        lanes = lax.iota(jnp.int32, 16)

        def hash_body(i, _):
            off = i * 16
            pos = off + lanes
            cur = idx_v[pl.ds(off, 16)]
            prev = prev_v[pl.ds(off, 16)]
            b = (prev * 10007 + cur) % BUCKETS
            b = jnp.where(pos % HIST == 0, 0, b)
            big_v[pl.ds(off, 16)] = b
            qid_v[pl.ds(off, 16)] = b >> 1
            return 0

        lax.fori_loop(0, TPW // 16, hash_body, 0)

        pltpu.sync_copy(big_v, big_hbm.at[pl.ds(base, TPW)])

        bufs = (buf_a, buf_b)
        sems = (sem_a, sem_b)

        def fire(c, buf, sem):
            cps = []
            for j in range(NGATHER):
                srow = c * OCHUNK + j * GCHUNK
                cps.append(pltpu.async_copy(
                    tbl_hbm.at[qid_v.at[pl.ds(srow, GCHUNK)]],
                    buf.at[pl.ds(j * GCHUNK, GCHUNK)],
                    sem))
            return cps

        inflight = {0: fire(0, bufs[0], sems[0])}
        for c in range(NOUT):
            if c + 1 < NOUT:
                inflight[c + 1] = fire(c + 1, bufs[(c + 1) % 2], sems[(c + 1) % 2])
            for cp in inflight.pop(c):
                cp.wait()
            pltpu.sync_copy(bufs[c % 2].at[:, pl.ds(0, D)],
                            rows_hbm.at[pl.ds(base + c * OCHUNK, OCHUNK),
                                        pl.ds(0, D)])

    return k(idx_flat, idx_prev, tblp)


def _f16_bits_to_f32(v):
    sign = (v & 0x8000) << 16
    mag = v & 0x7FFF
    bits = sign | ((mag << 13) + 0x38000000)
    bits = jnp.where((v & 0x7C00) == 0, sign, bits)
    return lax.bitcast_convert_type(bits, jnp.float32)


def _tc_project(rows32, bigs3, wt):
    def body(r_ref, b_ref, w_ref, o_ref):
        pcol = (b_ref[...] & 1).reshape(BT, 1)
        w = r_ref[...][:, :D]
        sel = jnp.where(pcol == 0, w & 0xFFFF, (w >> 16) & 0xFFFF)
        h = _f16_bits_to_f32(sel)
        o_ref[...] = jnp.dot(h, w_ref[...], preferred_element_type=jnp.float32)

    return pl.pallas_call(
        body,
        grid=(G,),
        in_specs=[
            pl.BlockSpec((BT, 128), lambda i: (i, 0)),
            pl.BlockSpec((1, 1, BT), lambda i: (i, 0, 0)),
            pl.BlockSpec((D, DIM), lambda i: (0, 0)),
        ],
        out_specs=pl.BlockSpec((BT, DIM), lambda i: (i, 0)),
        out_shape=jax.ShapeDtypeStruct((TOK, DIM), jnp.float32),
    )(rows32, bigs3, wt)


def kernel(idx, table, W):
    idx_flat = idx.reshape(-1)
    idx_prev = jnp.concatenate([idx_flat[:1], idx_flat[:-1]])
    tblp = _tc_prep(lax.bitcast_convert_type(table, jnp.bfloat16).T)
    rows32, bigs = _sc_hash_gather(idx_flat, idx_prev, tblp)
    out = _tc_project(rows32, bigs.reshape(G, 1, BT), W.T)
    return out.reshape(BATCH, HIST, DIM)


# speedup vs baseline: 5.7235x; 1.0364x over previous
import functools

import jax
import jax.numpy as jnp
from jax import lax
from jax.experimental import pallas as pl
from jax.experimental.pallas import tpu as pltpu
from jax.experimental.pallas import tpu_sc as plsc

BATCH = 1024
HIST = 200
BUCKETS = 1000000
D = 48
DIM = 512
TOK = BATCH * HIST

NC = 2
NS = 16
NW = NC * NS
HTOK = TOK // 2
TPW = HTOK // NW
GCHUNK = 128
OCHUNK = 128
NGATHER = OCHUNK // GCHUNK
NOUT = TPW // OCHUNK

BC = 2048
GP = -(-BUCKETS // BC)

BT = 2048
G = TOK // BT


def _tc_prep(tbl_bft):
    def body(t_ref, o_ref):
        x = t_ref[...]
        xt = x.T
        w = pltpu.bitcast(xt, jnp.int32)
        o_ref[:, :D] = w

    return pl.pallas_call(
        body,
        grid=(GP,),
        in_specs=[pl.BlockSpec((D, BC), lambda i: (0, i))],
        out_specs=pl.BlockSpec((BC // 2, 128), lambda i: (i, 0)),
        out_shape=jax.ShapeDtypeStruct((BUCKETS // 2, 128), jnp.int32),
    )(tbl_bft)


def _sc_hash_gather(idx_flat, idx_prev, tblp):
    mesh = plsc.VectorSubcoreMesh(core_axis_name="c", subcore_axis_name="s")

    @functools.partial(
        pl.kernel,
        mesh=mesh,
        out_type=(
            jax.ShapeDtypeStruct((HTOK, 128), jnp.int32),
            jax.ShapeDtypeStruct((HTOK,), jnp.int32),
        ),
        scratch_types=[
            pltpu.VMEM((TPW,), jnp.int32),
            pltpu.VMEM((TPW,), jnp.int32),
            pltpu.VMEM((TPW,), jnp.int32),
            pltpu.VMEM((TPW,), jnp.int32),
            pltpu.VMEM((OCHUNK, 128), jnp.int32),
            pltpu.VMEM((OCHUNK, 128), jnp.int32),
            pltpu.SemaphoreType.DMA,
            pltpu.SemaphoreType.DMA,
        ],
        compiler_params=pltpu.CompilerParams(use_tc_tiling_on_sc=False),
    )
    def k(idx_hbm, prev_hbm, tbl_hbm, rows_hbm, big_hbm,
          idx_v, prev_v, big_v, qid_v, buf_a, buf_b, sem_a, sem_b):
        wid = lax.axis_index("s") * NC + lax.axis_index("c")
        base = wid * TPW
        pltpu.sync_copy(idx_hbm.at[pl.ds(base, TPW)], idx_v)
        pltpu.sync_copy(prev_hbm.at[pl.ds(base, TPW)], prev_v)

        lanes = lax.iota(jnp.int32, 16)

        def hash_body(i, _):
            off = i * 16
            pos = off + lanes
            cur = idx_v[pl.ds(off, 16)]
            prev = prev_v[pl.ds(off, 16)]
            b = (prev * 10007 + cur) % BUCKETS
            b = jnp.where(pos % HIST == 0, 0, b)
            big_v[pl.ds(off, 16)] = b
            qid_v[pl.ds(off, 16)] = b >> 1
            return 0

        lax.fori_loop(0, TPW // 16, hash_body, 0)

        pltpu.sync_copy(big_v, big_hbm.at[pl.ds(base, TPW)])

        bufs = (buf_a, buf_b)
        sems = (sem_a, sem_b)

        def fire(c, buf, sem):
            cps = []
            for j in range(NGATHER):
                srow = c * OCHUNK + j * GCHUNK
                cps.append(pltpu.async_copy(
                    tbl_hbm.at[qid_v.at[pl.ds(srow, GCHUNK)]],
                    buf.at[pl.ds(j * GCHUNK, GCHUNK)],
                    sem))
            return cps

        inflight = {0: fire(0, bufs[0], sems[0])}
        for c in range(NOUT):
            if c + 1 < NOUT:
                inflight[c + 1] = fire(c + 1, bufs[(c + 1) % 2], sems[(c + 1) % 2])
            for cp in inflight.pop(c):
                cp.wait()
            pltpu.sync_copy(bufs[c % 2].at[:, pl.ds(0, D)],
                            rows_hbm.at[pl.ds(base + c * OCHUNK, OCHUNK),
                                        pl.ds(0, D)])

    return k(idx_flat, idx_prev, tblp)


def _f16_bits_to_f32(v):
    sign = (v & 0x8000) << 16
    mag = v & 0x7FFF
    bits = sign | ((mag << 13) + 0x38000000)
    bits = jnp.where((v & 0x7C00) == 0, sign, bits)
    return lax.bitcast_convert_type(bits, jnp.float32)


def _proj_body(r_ref, b_ref, w_ref, o_ref):
    pcol = (b_ref[...] & 1).reshape(BT, 1)
    w = r_ref[...][:, :D]
    sel = jnp.where(pcol == 0, w & 0xFFFF, (w >> 16) & 0xFFFF)
    h = _f16_bits_to_f32(sel)
    o_ref[...] = jnp.dot(h, w_ref[...], preferred_element_type=jnp.float32)


def _tc_project1(rows32, bigs3, wt):
    return pl.pallas_call(
        _proj_body,
        grid=(G // 2,),
        in_specs=[
            pl.BlockSpec((BT, 128), lambda i: (i, 0)),
            pl.BlockSpec((1, 1, BT), lambda i: (i, 0, 0)),
            pl.BlockSpec((D, DIM), lambda i: (0, 0)),
        ],
        out_specs=pl.BlockSpec((BT, DIM), lambda i: (i, 0)),
        out_shape=jax.ShapeDtypeStruct((TOK, DIM), jnp.float32),
    )(rows32, bigs3, wt)


def _tc_project2(rows32, bigs3, wt, outbuf):
    def body(r_ref, b_ref, w_ref, o_in_ref, o_ref):
        del o_in_ref
        _proj_body(r_ref, b_ref, w_ref, o_ref)

    return pl.pallas_call(
        body,
        grid=(G // 2,),
        in_specs=[
            pl.BlockSpec((BT, 128), lambda i: (i, 0)),
            pl.BlockSpec((1, 1, BT), lambda i: (i, 0, 0)),
            pl.BlockSpec((D, DIM), lambda i: (0, 0)),
            pl.BlockSpec(memory_space=pl.ANY),
        ],
        out_specs=pl.BlockSpec((BT, DIM), lambda i: (i + G // 2, 0)),
        out_shape=jax.ShapeDtypeStruct((TOK, DIM), jnp.float32),
        input_output_aliases={3: 0},
    )(rows32, bigs3, wt, outbuf)


def kernel(idx, table, W):
    idx_flat = idx.reshape(-1)
    idx_prev = jnp.concatenate([idx_flat[:1], idx_flat[:-1]])
    tblp = _tc_prep(lax.bitcast_convert_type(table, jnp.bfloat16).T)
    rows1, bigs1 = _sc_hash_gather(idx_flat[:HTOK], idx_prev[:HTOK], tblp)
    rows2, bigs2 = _sc_hash_gather(idx_flat[HTOK:], idx_prev[HTOK:], tblp)
    wt = W.T
    out1 = _tc_project1(rows1, bigs1.reshape(G // 2, 1, BT), wt)
    out = _tc_project2(rows2, bigs2.reshape(G // 2, 1, BT), wt, out1)
    return out.reshape(BATCH, HIST, DIM)


# speedup vs baseline: 5.7454x; 1.0038x over previous
import functools

import jax
import jax.numpy as jnp
from jax import lax
from jax.experimental import pallas as pl
from jax.experimental.pallas import tpu as pltpu
from jax.experimental.pallas import tpu_sc as plsc

BATCH = 1024
HIST = 200
BUCKETS = 1000000
D = 48
DIM = 512
TOK = BATCH * HIST

NC = 2
NS = 16
NW = NC * NS
NSPLIT = 4
HTOK = TOK // NSPLIT
TPW = HTOK // NW
GCHUNK = 80
OCHUNK = 160
NGATHER = OCHUNK // GCHUNK
NOUT = TPW // OCHUNK

BC = 2048
GP = -(-BUCKETS // BC)

BT = 2048
G = TOK // BT


def _tc_prep(tbl_bft):
    def body(t_ref, o_ref):
        x = t_ref[...]
        xt = x.T
        w = pltpu.bitcast(xt, jnp.int32)
        o_ref[:, :D] = w

    return pl.pallas_call(
        body,
        grid=(GP,),
        in_specs=[pl.BlockSpec((D, BC), lambda i: (0, i))],
        out_specs=pl.BlockSpec((BC // 2, 128), lambda i: (i, 0)),
        out_shape=jax.ShapeDtypeStruct((BUCKETS // 2, 128), jnp.int32),
    )(tbl_bft)


def _sc_hash_gather(idx_flat, idx_prev, tblp):
    mesh = plsc.VectorSubcoreMesh(core_axis_name="c", subcore_axis_name="s")

    @functools.partial(
        pl.kernel,
        mesh=mesh,
        out_type=(
            jax.ShapeDtypeStruct((HTOK, 128), jnp.int32),
            jax.ShapeDtypeStruct((HTOK,), jnp.int32),
        ),
        scratch_types=[
            pltpu.VMEM((TPW,), jnp.int32),
            pltpu.VMEM((TPW,), jnp.int32),
            pltpu.VMEM((TPW,), jnp.int32),
            pltpu.VMEM((TPW,), jnp.int32),
            pltpu.VMEM((OCHUNK, 128), jnp.int32),
            pltpu.VMEM((OCHUNK, 128), jnp.int32),
            pltpu.SemaphoreType.DMA,
            pltpu.SemaphoreType.DMA,
        ],
        compiler_params=pltpu.CompilerParams(use_tc_tiling_on_sc=False),
    )
    def k(idx_hbm, prev_hbm, tbl_hbm, rows_hbm, big_hbm,
          idx_v, prev_v, big_v, qid_v, buf_a, buf_b, sem_a, sem_b):
        wid = lax.axis_index("s") * NC + lax.axis_index("c")
        base = wid * TPW
        pltpu.sync_copy(idx_hbm.at[pl.ds(base, TPW)], idx_v)
        pltpu.sync_copy(prev_hbm.at[pl.ds(base, TPW)], prev_v)

        lanes = lax.iota(jnp.int32, 16)

        def hash_body(i, _):
            off = i * 16
            pos = off + lanes
            cur = idx_v[pl.ds(off, 16)]
            prev = prev_v[pl.ds(off, 16)]
            b = (prev * 10007 + cur) % BUCKETS
            b = jnp.where(pos % HIST == 0, 0, b)
            big_v[pl.ds(off, 16)] = b
            qid_v[pl.ds(off, 16)] = b >> 1
            return 0

        lax.fori_loop(0, TPW // 16, hash_body, 0)

        pltpu.sync_copy(big_v, big_hbm.at[pl.ds(base, TPW)])

        bufs = (buf_a, buf_b)
        sems = (sem_a, sem_b)

        def fire(c, buf, sem):
            cps = []
            for j in range(NGATHER):
                srow = c * OCHUNK + j * GCHUNK
                cps.append(pltpu.async_copy(
                    tbl_hbm.at[qid_v.at[pl.ds(srow, GCHUNK)]],
                    buf.at[pl.ds(j * GCHUNK, GCHUNK)],
                    sem))
            return cps

        inflight = {0: fire(0, bufs[0], sems[0])}
        for c in range(NOUT):
            if c + 1 < NOUT:
                inflight[c + 1] = fire(c + 1, bufs[(c + 1) % 2], sems[(c + 1) % 2])
            for cp in inflight.pop(c):
                cp.wait()
            pltpu.sync_copy(bufs[c % 2].at[:, pl.ds(0, D)],
                            rows_hbm.at[pl.ds(base + c * OCHUNK, OCHUNK),
                                        pl.ds(0, D)])

    return k(idx_flat, idx_prev, tblp)


def _f16_bits_to_f32(v):
    sign = (v & 0x8000) << 16
    mag = v & 0x7FFF
    bits = sign | ((mag << 13) + 0x38000000)
    bits = jnp.where((v & 0x7C00) == 0, sign, bits)
    return lax.bitcast_convert_type(bits, jnp.float32)


def _proj_body(r_ref, b_ref, w_ref, o_ref):
    pcol = (b_ref[...] & 1).reshape(BT, 1)
    w = r_ref[...][:, :D]
    sel = jnp.where(pcol == 0, w & 0xFFFF, (w >> 16) & 0xFFFF)
    h = _f16_bits_to_f32(sel)
    o_ref[...] = jnp.dot(h, w_ref[...], preferred_element_type=jnp.float32)


GS = G // NSPLIT


def _tc_project(rows32, bigs3, wt, split, outbuf):
    def body(r_ref, b_ref, w_ref, *rest):
        _proj_body(r_ref, b_ref, w_ref, rest[-1])

    in_specs = [
        pl.BlockSpec((BT, 128), lambda i: (i, 0)),
        pl.BlockSpec((1, 1, BT), lambda i: (i, 0, 0)),
        pl.BlockSpec((D, DIM), lambda i: (0, 0)),
    ]
    args = [rows32, bigs3, wt]
    aliases = {}
    if outbuf is not None:
        in_specs.append(pl.BlockSpec(memory_space=pl.ANY))
        args.append(outbuf)
        aliases = {3: 0}
    base = split * GS
    return pl.pallas_call(
        body,
        grid=(GS,),
        in_specs=in_specs,
        out_specs=pl.BlockSpec((BT, DIM), lambda i: (i + base, 0)),
        out_shape=jax.ShapeDtypeStruct((TOK, DIM), jnp.float32),
        input_output_aliases=aliases,
    )(*args)


def kernel(idx, table, W):
    idx_flat = idx.reshape(-1)
    idx_prev = jnp.concatenate([idx_flat[:1], idx_flat[:-1]])
    tblp = _tc_prep(lax.bitcast_convert_type(table, jnp.bfloat16).T)
    wt = W.T
    parts = []
    for s in range(NSPLIT):
        lo = s * HTOK
        parts.append(_sc_hash_gather(
            idx_flat[lo:lo + HTOK], idx_prev[lo:lo + HTOK], tblp))
    out = None
    for s, (rows_s, bigs_s) in enumerate(parts):
        out = _tc_project(rows_s, bigs_s.reshape(GS, 1, BT), wt, s, out)
    return out.reshape(BATCH, HIST, DIM)
